# baseline (speedup 1.0000x reference)
import functools

import jax
import jax.numpy as jnp
from jax import lax
from jax.experimental import pallas as pl
from jax.experimental.pallas import tpu as pltpu
from jax.experimental.pallas import tpu_sc as plsc

N_U = 25000
N_I = 25000
N = N_U + N_I
D = 64
E = 800000
EH = E // 2
TAU = 0.2
REG_LAMBDA = 1e-4
SSL_LAMBDA = 0.1
B = 4096

NTILE = 16
NCORE = 2
CH = 256
PT = 25088
NCHUNK = PT // CH
BLK = 14
NBLK = 14
PE = NTILE * PT
PAD = PE - EH
ROWS2D = 2 * PE // 128
ACC_ROWS = PT
STRIPE = ACC_ROWS // NTILE

_mesh = plsc.VectorSubcoreMesh(core_axis_name="c", subcore_axis_name="s")


HR = 208


@functools.partial(
    pl.kernel,
    out_type=jax.ShapeDtypeStruct((NCORE * HR, 128), jnp.float32),
    mesh=_mesh,
    compiler_params=pltpu.CompilerParams(use_tc_tiling_on_sc=False,
                                         needs_layout_passes=False),
    scratch_types=[
        pltpu.VMEM_SHARED((HR, 128), jnp.float32),
        pltpu.VMEM((HR, 128), jnp.float32),
        pltpu.VMEM((HR,), jnp.int32),
        pltpu.VMEM((BLK, 128), jnp.int32),
        pltpu.VMEM((13, 128), jnp.float32),
    ],
)
def _deg_kernel(rows2d, deg_out, hist, histv, ids, rbuf, vbuf):
    c = lax.axis_index("c")
    s = lax.axis_index("s")
    ones16 = jnp.ones((16,), jnp.float32)

    def zfill(i, carry):
        for g in range(8):
            histv[i, pl.ds(g * 16, 16)] = jnp.zeros((16,), jnp.float32)
        return carry

    lax.fori_loop(0, HR, zfill, 0)
    for g in range(HR // 16):
        ids[pl.ds(g * 16, 16)] = lax.iota(jnp.int32, 16) + (g * 16)
    for i in range(13):
        for g in range(8):
            vbuf[i, pl.ds(g * 16, 16)] = jnp.zeros((16,), jnp.float32)
    pltpu.sync_copy(vbuf, hist.at[pl.ds(s * 13, 13)])
    plsc.subcore_barrier()

    base2d = c * (PE // 128) + s * (PT // 128)

    def blk(k, carry):
        pltpu.sync_copy(rows2d.at[pl.ds(base2d + k * BLK, BLK)], rbuf)
        for j in range(BLK):
            for g in range(8):
                idx16 = rbuf[j, pl.ds(g * 16, 16)]
                plsc.addupdate_scatter(
                    histv,
                    [lax.shift_right_logical(idx16, 7),
                     lax.bitwise_and(idx16, 127)],
                    ones16)
        return carry

    lax.fori_loop(0, NBLK, blk, 0)
    pltpu.sync_copy(histv, hist.at[ids], add=True)
    plsc.subcore_barrier()
    pltpu.sync_copy(hist.at[pl.ds(s * 13, 13)], vbuf)
    pltpu.sync_copy(vbuf, deg_out.at[pl.ds(c * HR + s * 13, 13)])


_PROP_SCRATCH = [
    pltpu.VMEM_SHARED((ACC_ROWS, D), jnp.float32),
    pltpu.VMEM((BLK, 128), jnp.int32),
    pltpu.VMEM((BLK, 128), jnp.int32),
    pltpu.VMEM((BLK, 128), jnp.int32),
    pltpu.VMEM((BLK, 128), jnp.int32),
    pltpu.VMEM((128, D), jnp.float32),
    pltpu.VMEM((128, D), jnp.float32),
    pltpu.VMEM((128, 16), jnp.float32),
    pltpu.VMEM((1, 128), jnp.int32),
    pltpu.SemaphoreType.DMA,
    pltpu.SemaphoreType.DMA,
    pltpu.SemaphoreType.DMA,
    pltpu.SemaphoreType.DMA,
]


def _zero_acc(acc, ga, s):
    def zfill(i, carry):
        for g in range(D // 16):
            ga[i, pl.ds(g * 16, 16)] = jnp.zeros((16,), jnp.float32)
        return carry

    lax.fori_loop(0, 128, zfill, 0)

    def zcopy(i, carry):
        pltpu.sync_copy(ga, acc.at[pl.ds(s * STRIPE + i * 128, 128)])
        return carry

    lax.fori_loop(0, STRIPE // 128, zcopy, 0)
    pltpu.sync_copy(ga.at[pl.ds(0, STRIPE - (STRIPE // 128) * 128)],
                    acc.at[pl.ds(s * STRIPE + (STRIPE // 128) * 128,
                                 STRIPE - (STRIPE // 128) * 128)])


def _edge_pipeline(y, rows2d, cols2d, acc, r0, c0, r1, c1, ga, gb,
                   rdum, sema, semb, ssema, ssemb, c, s):
    base2d = c * (PE // 128) + s * (PT // 128)
    bufs = ((ga, sema, ssema), (gb, semb, ssemb))

    def fire(colref, gbuf, sem):
        pltpu.async_copy(y.at[colref], gbuf, sem)

    def wait_scatter(gbuf, ssem):
        pltpu.make_async_copy(gbuf, acc.at[pl.ds(0, 128)], ssem).wait()

    def drain_scatter(gbuf, sem, ssem, rowref):
        pltpu.make_async_copy(y.at[pl.ds(0, 128)], gbuf, sem).wait()
        pltpu.async_copy(gbuf, acc.at[rowref], ssem, add=True)

    pltpu.sync_copy(rows2d.at[pl.ds(base2d, BLK)], r0)
    pltpu.sync_copy(cols2d.at[pl.ds(base2d, BLK)], c0)
    for g in range(8):
        rdum[0, pl.ds(g * 16, 16)] = jnp.full((16,), N_U, jnp.int32)
    pltpu.async_copy(ga, acc.at[rdum.at[0]], ssema, add=True)
    pltpu.async_copy(gb, acc.at[rdum.at[0]], ssemb, add=True)
    fire(c0.at[0], ga, sema)

    def blockpair(m, carry):
        off1 = base2d + (2 * m + 1) * BLK
        pltpu.sync_copy(rows2d.at[pl.ds(off1, BLK)], r1)
        pltpu.sync_copy(cols2d.at[pl.ds(off1, BLK)], c1)
        for i in range(BLK):
            cur, nxt = bufs[i % 2], bufs[(i + 1) % 2]
            nxtcol = c0.at[i + 1] if i < BLK - 1 else c1.at[0]
            wait_scatter(nxt[0], nxt[2])
            fire(nxtcol, nxt[0], nxt[1])
            drain_scatter(cur[0], cur[1], cur[2], r0.at[i])
        off2 = base2d + (2 * m + 2) * BLK

        @pl.when(m < NBLK // 2 - 1)
        def _():
            pltpu.sync_copy(rows2d.at[pl.ds(off2, BLK)], r0)
            pltpu.sync_copy(cols2d.at[pl.ds(off2, BLK)], c0)

        for i in range(BLK):
            cur, nxt = bufs[i % 2], bufs[(i + 1) % 2]
            if i < BLK - 1:
                wait_scatter(nxt[0], nxt[2])
                fire(c1.at[i + 1], nxt[0], nxt[1])
            else:
                @pl.when(m < NBLK // 2 - 1)
                def _():
                    wait_scatter(nxt[0], nxt[2])
                    fire(c0.at[0], nxt[0], nxt[1])
            drain_scatter(cur[0], cur[1], cur[2], r1.at[i])
        return carry

    lax.fori_loop(0, NBLK // 2, blockpair, 0)
    wait_scatter(ga, ssema)
    wait_scatter(gb, ssemb)
    wait_scatter(ga, ssema)


@functools.partial(
    pl.kernel,
    out_type=jax.ShapeDtypeStruct((N, D), jnp.float32),
    mesh=_mesh,
    compiler_params=pltpu.CompilerParams(use_tc_tiling_on_sc=False),
    scratch_types=_PROP_SCRATCH,
)
def _prop1_kernel(y, rows2d, cols2d, degbc, y1,
                  acc, r0, c0, r1, c1, ga, gb, dbuf, rdum,
                  sema, semb, ssema, ssemb):
    c = lax.axis_index("c")
    s = lax.axis_index("s")
    _zero_acc(acc, ga, s)
    plsc.subcore_barrier()
    _edge_pipeline(y, rows2d, cols2d, acc, r0, c0, r1, c1, ga, gb,
                   rdum, sema, semb, ssema, ssemb, c, s)
    plsc.subcore_barrier()

    def wb(i, carry):
        ck = s + i * NTILE

        @pl.when(ck < 200)
        def _():
            off = ck * 125
            pltpu.sync_copy(acc.at[pl.ds(off, 125)], ga.at[pl.ds(0, 125)])
            pltpu.sync_copy(degbc.at[pl.ds(c * N_U + off, 125)],
                            dbuf.at[pl.ds(0, 125)])

            def scale_row(r, carry2):
                inv = dbuf[r, pl.ds(0, 16)][1]
                for g in range(D // 16):
                    ga[r, pl.ds(g * 16, 16)] = ga[r, pl.ds(g * 16, 16)] * inv
                return carry2

            lax.fori_loop(0, 125, scale_row, 0)
            pltpu.sync_copy(ga.at[pl.ds(0, 125)], y1.at[pl.ds(c * N_U + off, 125)])

        return carry

    lax.fori_loop(0, 200 // NTILE + 1, wb, 0)


@functools.partial(
    pl.kernel,
    out_type=(
        jax.ShapeDtypeStruct((3 * B, D), jnp.float32),
        jax.ShapeDtypeStruct((3 * B, D), jnp.float32),
        jax.ShapeDtypeStruct((3 * B, 16), jnp.float32),
        jax.ShapeDtypeStruct((3 * B, D), jnp.float32),
    ),
    mesh=_mesh,
    compiler_params=pltpu.CompilerParams(use_tc_tiling_on_sc=False),
    scratch_types=_PROP_SCRATCH + [
        pltpu.VMEM((3, 128), jnp.int32),
        pltpu.VMEM((4, 128), jnp.int32),
    ],
)
def _prop2_kernel(y, rows2d, cols2d, e0, degbc, idxn2d, idxacc2d,
                  e0g, y1g, degg, z2g,
                  acc, r0, c0, r1, c1, ga, gb, dbuf, rdum,
                  sema, semb, ssema, ssemb, ibuf, ibacc):
    c = lax.axis_index("c")
    s = lax.axis_index("s")
    _zero_acc(acc, ga, s)
    plsc.subcore_barrier()
    _edge_pipeline(y, rows2d, cols2d, acc, r0, c0, r1, c1, ga, gb,
                   rdum, sema, semb, ssema, ssemb, c, s)

    w = c * NTILE + s
    pltpu.sync_copy(idxn2d.at[pl.ds(w * 3, 3)], ibuf)
    for j in range(3):
        r = w * 3 + j
        for src, dst in ((e0, e0g), (y, y1g)):
            pltpu.async_copy(src.at[ibuf.at[j]], ga, sema).wait()
            pltpu.sync_copy(ga, dst.at[pl.ds(r * 128, 128)])
        pltpu.async_copy(degbc.at[ibuf.at[j]], dbuf, sema).wait()
        pltpu.sync_copy(dbuf, degg.at[pl.ds(r * 128, 128)])

    plsc.subcore_barrier()

    @pl.when(c == 0)
    def _():
        pltpu.sync_copy(idxacc2d.at[pl.ds(2 * s, 2)], ibacc.at[pl.ds(0, 2)])
        for j in range(2):
            pltpu.sync_copy(acc.at[ibacc.at[j]], ga)
            pltpu.sync_copy(ga, z2g.at[pl.ds((2 * s + j) * 128, 128)])

    @pl.when(c == 1)
    def _():
        pltpu.sync_copy(idxacc2d.at[pl.ds(32 + 4 * s, 4)], ibacc)
        for j in range(4):
            pltpu.sync_copy(acc.at[ibacc.at[j]], ga)
            pltpu.sync_copy(ga, z2g.at[pl.ds((32 + 4 * s + j) * 128, 128)])


def _scale_a_body(e_ref, d_ref, o_ref, o2_ref):
    dd = jnp.maximum(d_ref[...], 1.0)
    o_ref[...] = e_ref[...] * lax.rsqrt(dd)
    o2_ref[...] = jnp.concatenate(
        [d_ref[...], 1.0 / dd, jnp.broadcast_to(d_ref[...], (400, 14))], axis=1)


def _scale_a(x, deg2d):
    return pl.pallas_call(
        _scale_a_body,
        grid=(125,),
        in_specs=[
            pl.BlockSpec((400, D), lambda i: (i, 0)),
            pl.BlockSpec((400, 1), lambda i: (i, 0)),
        ],
        out_specs=(
            pl.BlockSpec((400, D), lambda i: (i, 0)),
            pl.BlockSpec((400, 16), lambda i: (i, 0)),
        ),
        out_shape=(
            jax.ShapeDtypeStruct((N, D), jnp.float32),
            jax.ShapeDtypeStruct((N, 16), jnp.float32),
        ),
        compiler_params=pltpu.CompilerParams(dimension_semantics=("arbitrary",)),
    )(x, deg2d)


def _loss_body(eu, ep, en, epf, z1u, z1p, z1n, z1pf, z2u, z2p, z2n, z2pf,
               du, dp, dn, dpf, out_ref, acc):
    i = pl.program_id(0)

    @pl.when(i == 0)
    def _():
        acc[0] = 0.0
        acc[1] = 0.0
        acc[2] = 0.0

    def fin(e, y1v, z2v, dd):
        dm = jnp.maximum(dd[:, 0:1], 1.0)
        sc = lax.rsqrt(dm)
        return (e + sc * (y1v * dm + z2v)) * (1.0 / 3.0)

    u = fin(eu[...], z1u[...], z2u[...], du[...])
    p = fin(ep[...], z1p[...], z2p[...], dp[...])
    nn = fin(en[...], z1n[...], z2n[...], dn[...])
    pf = fin(epf[...], z1pf[...], z2pf[...], dpf[...])

    pos_s = jnp.sum(u * p, axis=-1)
    neg_s = jnp.sum(u * nn, axis=-1)
    x = neg_s - pos_s
    bpr = jnp.sum(jnp.maximum(x, 0.0) + jnp.log(1.0 + jnp.exp(-jnp.abs(x))))
    reg = jnp.sum(eu[...] ** 2) + jnp.sum(ep[...] ** 2) + jnp.sum(en[...] ** 2)

    un = u / jnp.maximum(jnp.sqrt(jnp.sum(u * u, axis=-1, keepdims=True)), 1e-8)
    pnm = p / jnp.maximum(jnp.sqrt(jnp.sum(p * p, axis=-1, keepdims=True)), 1e-8)
    pnf = pf / jnp.maximum(jnp.sqrt(jnp.sum(pf * pf, axis=-1, keepdims=True)), 1e-8)
    pos_score = jnp.sum(un * pnm, axis=-1) * (1.0 / TAU)
    logits = lax.dot_general(un, pnf, (((1,), (1,)), ((), ())),
                             preferred_element_type=jnp.float32) * (1.0 / TAU)
    m = jnp.max(logits, axis=1)
    ttl = jnp.log(jnp.sum(jnp.exp(logits - m[:, None]), axis=1)) + m
    na = jnp.sum(ttl - pos_score)

    acc[0] += bpr
    acc[1] += reg
    acc[2] += na

    @pl.when(i == 7)
    def _():
        r = lax.broadcasted_iota(jnp.int32, (8, 128), 0)
        cc = lax.broadcasted_iota(jnp.int32, (8, 128), 1)
        vals = jnp.where((r == 0) & (cc == 0), acc[0] / B, 0.0)
        vals = vals + jnp.where((r == 0) & (cc == 1), REG_LAMBDA * 0.5 * acc[1] / B, 0.0)
        vals = vals + jnp.where((r == 0) & (cc == 2), SSL_LAMBDA * acc[2] / B, 0.0)
        out_ref[...] = vals


def _losses(e0g, z1g, z2g, degg):
    bu = pl.BlockSpec((512, D), lambda i: (i, 0))
    bp = pl.BlockSpec((512, D), lambda i: (8 + i, 0))
    bn = pl.BlockSpec((512, D), lambda i: (16 + i, 0))
    bpf = pl.BlockSpec((B, D), lambda i: (1, 0))
    du = pl.BlockSpec((512, 16), lambda i: (i, 0))
    dp = pl.BlockSpec((512, 16), lambda i: (8 + i, 0))
    dn = pl.BlockSpec((512, 16), lambda i: (16 + i, 0))
    dpf = pl.BlockSpec((B, 16), lambda i: (1, 0))
    return pl.pallas_call(
        _loss_body,
        grid=(8,),
        in_specs=[bu, bp, bn, bpf] * 3 + [du, dp, dn, dpf],
        out_specs=pl.BlockSpec((8, 128), lambda i: (0, 0)),
        out_shape=jax.ShapeDtypeStruct((8, 128), jnp.float32),
        scratch_shapes=[pltpu.SMEM((4,), jnp.float32)],
        compiler_params=pltpu.CompilerParams(dimension_semantics=("arbitrary",)),
    )(e0g, e0g, e0g, e0g, z1g, z1g, z1g, z1g, z2g, z2g, z2g, z2g,
      degg, degg, degg, degg)


def kernel(user_table, item_table, adj_val, adj_row, adj_col, user, positive, negative):
    del adj_val
    e0 = jnp.concatenate([user_table, item_table], axis=0)

    pad_r = jnp.full((PAD,), N_U, jnp.int32)
    pad_c = jnp.zeros((PAD,), jnp.int32)
    rows2d = jnp.concatenate(
        [adj_row[:EH], pad_r, adj_row[EH:] - N_U, pad_r]).reshape(ROWS2D, 128)
    cols2d = jnp.concatenate(
        [adj_col[:EH], pad_c, adj_col[EH:], pad_c]).reshape(ROWS2D, 128)

    degflat = _deg_kernel(rows2d).reshape(-1)
    deg2d = jnp.concatenate(
        [degflat[:N_U], degflat[HR * 128:HR * 128 + N_U]])[:, None]

    y0, degbc = _scale_a(e0, deg2d)
    y1 = _prop1_kernel(y0, rows2d, cols2d, degbc)

    idxn2d = jnp.concatenate(
        [user, positive + N_U, negative + N_U]).astype(jnp.int32).reshape(96, 128)
    idxacc2d = jnp.concatenate(
        [user, positive, negative]).astype(jnp.int32).reshape(96, 128)
    e0g, y1g, degg, z2g = _prop2_kernel(
        y1, rows2d, cols2d, e0, degbc, idxn2d, idxacc2d)

    out = _losses(e0g, y1g, z2g, degg)
    return out[0, :3]


# speedup vs baseline: 1.0530x; 1.0530x over previous
import functools

import jax
import jax.numpy as jnp
from jax import lax
from jax.experimental import pallas as pl
from jax.experimental.pallas import tpu as pltpu
from jax.experimental.pallas import tpu_sc as plsc

N_U = 25000
N_I = 25000
N = N_U + N_I
D = 64
E = 800000
EH = E // 2
TAU = 0.2
REG_LAMBDA = 1e-4
SSL_LAMBDA = 0.1
B = 4096

NTILE = 16
NCORE = 2
CH = 256
PT = 25088
NCHUNK = PT // CH
BLK = 14
NBLK = 14
PE = NTILE * PT
PAD = PE - EH
ROWS2D = 2 * PE // 128
ACC_ROWS = PT
STRIPE = ACC_ROWS // NTILE

_mesh = plsc.VectorSubcoreMesh(core_axis_name="c", subcore_axis_name="s")


HR = 208


@functools.partial(
    pl.kernel,
    out_type=jax.ShapeDtypeStruct((NCORE * HR, 128), jnp.float32),
    mesh=_mesh,
    compiler_params=pltpu.CompilerParams(use_tc_tiling_on_sc=False,
                                         needs_layout_passes=False),
    scratch_types=[
        pltpu.VMEM_SHARED((HR, 128), jnp.float32),
        pltpu.VMEM((HR, 128), jnp.float32),
        pltpu.VMEM((HR,), jnp.int32),
        pltpu.VMEM((BLK, 128), jnp.int32),
        pltpu.VMEM((13, 128), jnp.float32),
    ],
)
def _deg_kernel(rows2d, deg_out, hist, histv, ids, rbuf, vbuf):
    c = lax.axis_index("c")
    s = lax.axis_index("s")
    ones16 = jnp.ones((16,), jnp.float32)

    def zfill(i, carry):
        for g in range(8):
            histv[i, pl.ds(g * 16, 16)] = jnp.zeros((16,), jnp.float32)
        return carry

    lax.fori_loop(0, HR, zfill, 0)
    for g in range(HR // 16):
        ids[pl.ds(g * 16, 16)] = lax.iota(jnp.int32, 16) + (g * 16)
    for i in range(13):
        for g in range(8):
            vbuf[i, pl.ds(g * 16, 16)] = jnp.zeros((16,), jnp.float32)
    pltpu.sync_copy(vbuf, hist.at[pl.ds(s * 13, 13)])
    plsc.subcore_barrier()

    base2d = c * (PE // 128) + s * (PT // 128)

    def blk(k, carry):
        pltpu.sync_copy(rows2d.at[pl.ds(base2d + k * BLK, BLK)], rbuf)
        for j in range(BLK):
            for g in range(8):
                idx16 = rbuf[j, pl.ds(g * 16, 16)]
                plsc.addupdate_scatter(
                    histv,
                    [lax.shift_right_logical(idx16, 7),
                     lax.bitwise_and(idx16, 127)],
                    ones16)
        return carry

    lax.fori_loop(0, NBLK, blk, 0)
    pltpu.sync_copy(histv, hist.at[ids], add=True)
    plsc.subcore_barrier()
    pltpu.sync_copy(hist.at[pl.ds(s * 13, 13)], vbuf)
    pltpu.sync_copy(vbuf, deg_out.at[pl.ds(c * HR + s * 13, 13)])


_PROP_SCRATCH = [
    pltpu.VMEM_SHARED((ACC_ROWS, D), jnp.float32),
    pltpu.VMEM((BLK, 128), jnp.int32),
    pltpu.VMEM((BLK, 128), jnp.int32),
    pltpu.VMEM((BLK, 128), jnp.int32),
    pltpu.VMEM((BLK, 128), jnp.int32),
    pltpu.VMEM((128, D), jnp.float32),
    pltpu.VMEM((128, D), jnp.float32),
    pltpu.VMEM((128, 16), jnp.float32),
    pltpu.SemaphoreType.DMA,
    pltpu.SemaphoreType.DMA,
]


def _zero_acc(acc, ga, s):
    def zfill(i, carry):
        for g in range(D // 16):
            ga[i, pl.ds(g * 16, 16)] = jnp.zeros((16,), jnp.float32)
        return carry

    lax.fori_loop(0, 128, zfill, 0)

    def zcopy(i, carry):
        pltpu.sync_copy(ga, acc.at[pl.ds(s * STRIPE + i * 128, 128)])
        return carry

    lax.fori_loop(0, STRIPE // 128, zcopy, 0)
    pltpu.sync_copy(ga.at[pl.ds(0, STRIPE - (STRIPE // 128) * 128)],
                    acc.at[pl.ds(s * STRIPE + (STRIPE // 128) * 128,
                                 STRIPE - (STRIPE // 128) * 128)])


def _edge_pipeline(y, rows2d, cols2d, acc, r0, c0, r1, c1, ga, gb, sema, semb, c, s):
    base2d = c * (PE // 128) + s * (PT // 128)
    bufs = ((ga, sema), (gb, semb))

    def fire(colref, gbuf, sem):
        pltpu.async_copy(y.at[colref], gbuf, sem)

    def drain_scatter(gbuf, sem, rowref):
        pltpu.make_async_copy(y.at[pl.ds(0, 128)], gbuf, sem).wait()
        pltpu.sync_copy(gbuf, acc.at[rowref], add=True)

    pltpu.sync_copy(rows2d.at[pl.ds(base2d, BLK)], r0)
    pltpu.sync_copy(cols2d.at[pl.ds(base2d, BLK)], c0)
    fire(c0.at[0], ga, sema)

    def blockpair(m, carry):
        off1 = base2d + (2 * m + 1) * BLK
        pltpu.sync_copy(rows2d.at[pl.ds(off1, BLK)], r1)
        pltpu.sync_copy(cols2d.at[pl.ds(off1, BLK)], c1)
        for i in range(BLK):
            cur, nxt = bufs[i % 2], bufs[(i + 1) % 2]
            nxtcol = c0.at[i + 1] if i < BLK - 1 else c1.at[0]
            fire(nxtcol, nxt[0], nxt[1])
            drain_scatter(cur[0], cur[1], r0.at[i])
        off2 = base2d + (2 * m + 2) * BLK

        @pl.when(m < NBLK // 2 - 1)
        def _():
            pltpu.sync_copy(rows2d.at[pl.ds(off2, BLK)], r0)
            pltpu.sync_copy(cols2d.at[pl.ds(off2, BLK)], c0)

        for i in range(BLK):
            cur, nxt = bufs[i % 2], bufs[(i + 1) % 2]
            if i < BLK - 1:
                fire(c1.at[i + 1], nxt[0], nxt[1])
            else:
                @pl.when(m < NBLK // 2 - 1)
                def _():
                    fire(c0.at[0], nxt[0], nxt[1])
            drain_scatter(cur[0], cur[1], r1.at[i])
        return carry

    lax.fori_loop(0, NBLK // 2, blockpair, 0)


@functools.partial(
    pl.kernel,
    out_type=jax.ShapeDtypeStruct((N, D), jnp.float32),
    mesh=_mesh,
    compiler_params=pltpu.CompilerParams(use_tc_tiling_on_sc=False),
    scratch_types=_PROP_SCRATCH,
)
def _prop1_kernel(y, rows2d, cols2d, degbc, y1,
                  acc, r0, c0, r1, c1, ga, gb, dbuf, sema, semb):
    c = lax.axis_index("c")
    s = lax.axis_index("s")
    _zero_acc(acc, ga, s)
    plsc.subcore_barrier()
    _edge_pipeline(y, rows2d, cols2d, acc, r0, c0, r1, c1, ga, gb, sema, semb, c, s)
    plsc.subcore_barrier()

    def wb(i, carry):
        ck = s + i * NTILE

        @pl.when(ck < 200)
        def _():
            off = ck * 125
            pltpu.sync_copy(acc.at[pl.ds(off, 125)], ga.at[pl.ds(0, 125)])
            pltpu.sync_copy(degbc.at[pl.ds(c * N_U + off, 125)],
                            dbuf.at[pl.ds(0, 125)])

            def scale_row(r, carry2):
                inv = dbuf[r, pl.ds(0, 16)][1]
                for g in range(D // 16):
                    ga[r, pl.ds(g * 16, 16)] = ga[r, pl.ds(g * 16, 16)] * inv
                return carry2

            lax.fori_loop(0, 125, scale_row, 0)
            pltpu.sync_copy(ga.at[pl.ds(0, 125)], y1.at[pl.ds(c * N_U + off, 125)])

        return carry

    lax.fori_loop(0, 200 // NTILE + 1, wb, 0)


@functools.partial(
    pl.kernel,
    out_type=(
        jax.ShapeDtypeStruct((3 * B, D), jnp.float32),
        jax.ShapeDtypeStruct((3 * B, D), jnp.float32),
        jax.ShapeDtypeStruct((3 * B, 16), jnp.float32),
        jax.ShapeDtypeStruct((3 * B, D), jnp.float32),
    ),
    mesh=_mesh,
    compiler_params=pltpu.CompilerParams(use_tc_tiling_on_sc=False),
    scratch_types=_PROP_SCRATCH + [
        pltpu.VMEM((3, 128), jnp.int32),
        pltpu.VMEM((4, 128), jnp.int32),
    ],
)
def _prop2_kernel(y, rows2d, cols2d, e0, degbc, idxn2d, idxacc2d,
                  e0g, y1g, degg, z2g,
                  acc, r0, c0, r1, c1, ga, gb, dbuf, sema, semb, ibuf, ibacc):
    c = lax.axis_index("c")
    s = lax.axis_index("s")
    _zero_acc(acc, ga, s)
    plsc.subcore_barrier()
    _edge_pipeline(y, rows2d, cols2d, acc, r0, c0, r1, c1, ga, gb, sema, semb, c, s)

    w = c * NTILE + s
    pltpu.sync_copy(idxn2d.at[pl.ds(w * 3, 3)], ibuf)
    for j in range(3):
        r = w * 3 + j
        for src, dst in ((e0, e0g), (y, y1g)):
            pltpu.async_copy(src.at[ibuf.at[j]], ga, sema).wait()
            pltpu.sync_copy(ga, dst.at[pl.ds(r * 128, 128)])
        pltpu.async_copy(degbc.at[ibuf.at[j]], dbuf, sema).wait()
        pltpu.sync_copy(dbuf, degg.at[pl.ds(r * 128, 128)])

    plsc.subcore_barrier()

    @pl.when(c == 0)
    def _():
        pltpu.sync_copy(idxacc2d.at[pl.ds(2 * s, 2)], ibacc.at[pl.ds(0, 2)])
        for j in range(2):
            pltpu.sync_copy(acc.at[ibacc.at[j]], ga)
            pltpu.sync_copy(ga, z2g.at[pl.ds((2 * s + j) * 128, 128)])

    @pl.when(c == 1)
    def _():
        pltpu.sync_copy(idxacc2d.at[pl.ds(32 + 4 * s, 4)], ibacc)
        for j in range(4):
            pltpu.sync_copy(acc.at[ibacc.at[j]], ga)
            pltpu.sync_copy(ga, z2g.at[pl.ds((32 + 4 * s + j) * 128, 128)])


def _scale_a_body(e_ref, d_ref, o_ref, o2_ref):
    dd = jnp.maximum(d_ref[...], 1.0)
    o_ref[...] = e_ref[...] * lax.rsqrt(dd)
    o2_ref[...] = jnp.concatenate(
        [d_ref[...], 1.0 / dd, jnp.broadcast_to(d_ref[...], (400, 14))], axis=1)


def _scale_a(x, deg2d):
    return pl.pallas_call(
        _scale_a_body,
        grid=(125,),
        in_specs=[
            pl.BlockSpec((400, D), lambda i: (i, 0)),
            pl.BlockSpec((400, 1), lambda i: (i, 0)),
        ],
        out_specs=(
            pl.BlockSpec((400, D), lambda i: (i, 0)),
            pl.BlockSpec((400, 16), lambda i: (i, 0)),
        ),
        out_shape=(
            jax.ShapeDtypeStruct((N, D), jnp.float32),
            jax.ShapeDtypeStruct((N, 16), jnp.float32),
        ),
        compiler_params=pltpu.CompilerParams(dimension_semantics=("arbitrary",)),
    )(x, deg2d)


def _loss_body(eu, ep, en, epf, z1u, z1p, z1n, z1pf, z2u, z2p, z2n, z2pf,
               du, dp, dn, dpf, out_ref, acc):
    i = pl.program_id(0)

    @pl.when(i == 0)
    def _():
        acc[0] = 0.0
        acc[1] = 0.0
        acc[2] = 0.0

    def fin(e, y1v, z2v, dd):
        dm = jnp.maximum(dd[:, 0:1], 1.0)
        sc = lax.rsqrt(dm)
        return (e + sc * (y1v * dm + z2v)) * (1.0 / 3.0)

    u = fin(eu[...], z1u[...], z2u[...], du[...])
    p = fin(ep[...], z1p[...], z2p[...], dp[...])
    nn = fin(en[...], z1n[...], z2n[...], dn[...])
    pf = fin(epf[...], z1pf[...], z2pf[...], dpf[...])

    pos_s = jnp.sum(u * p, axis=-1)
    neg_s = jnp.sum(u * nn, axis=-1)
    x = neg_s - pos_s
    bpr = jnp.sum(jnp.maximum(x, 0.0) + jnp.log(1.0 + jnp.exp(-jnp.abs(x))))
    reg = jnp.sum(eu[...] ** 2) + jnp.sum(ep[...] ** 2) + jnp.sum(en[...] ** 2)

    un = u / jnp.maximum(jnp.sqrt(jnp.sum(u * u, axis=-1, keepdims=True)), 1e-8)
    pnm = p / jnp.maximum(jnp.sqrt(jnp.sum(p * p, axis=-1, keepdims=True)), 1e-8)
    pnf = pf / jnp.maximum(jnp.sqrt(jnp.sum(pf * pf, axis=-1, keepdims=True)), 1e-8)
    pos_score = jnp.sum(un * pnm, axis=-1) * (1.0 / TAU)
    logits = lax.dot_general(un, pnf, (((1,), (1,)), ((), ())),
                             preferred_element_type=jnp.float32) * (1.0 / TAU)
    m = jnp.max(logits, axis=1)
    ttl = jnp.log(jnp.sum(jnp.exp(logits - m[:, None]), axis=1)) + m
    na = jnp.sum(ttl - pos_score)

    acc[0] += bpr
    acc[1] += reg
    acc[2] += na

    @pl.when(i == 7)
    def _():
        r = lax.broadcasted_iota(jnp.int32, (8, 128), 0)
        cc = lax.broadcasted_iota(jnp.int32, (8, 128), 1)
        vals = jnp.where((r == 0) & (cc == 0), acc[0] / B, 0.0)
        vals = vals + jnp.where((r == 0) & (cc == 1), REG_LAMBDA * 0.5 * acc[1] / B, 0.0)
        vals = vals + jnp.where((r == 0) & (cc == 2), SSL_LAMBDA * acc[2] / B, 0.0)
        out_ref[...] = vals


def _losses(e0g, z1g, z2g, degg):
    bu = pl.BlockSpec((512, D), lambda i: (i, 0))
    bp = pl.BlockSpec((512, D), lambda i: (8 + i, 0))
    bn = pl.BlockSpec((512, D), lambda i: (16 + i, 0))
    bpf = pl.BlockSpec((B, D), lambda i: (1, 0))
    du = pl.BlockSpec((512, 16), lambda i: (i, 0))
    dp = pl.BlockSpec((512, 16), lambda i: (8 + i, 0))
    dn = pl.BlockSpec((512, 16), lambda i: (16 + i, 0))
    dpf = pl.BlockSpec((B, 16), lambda i: (1, 0))
    return pl.pallas_call(
        _loss_body,
        grid=(8,),
        in_specs=[bu, bp, bn, bpf] * 3 + [du, dp, dn, dpf],
        out_specs=pl.BlockSpec((8, 128), lambda i: (0, 0)),
        out_shape=jax.ShapeDtypeStruct((8, 128), jnp.float32),
        scratch_shapes=[pltpu.SMEM((4,), jnp.float32)],
        compiler_params=pltpu.CompilerParams(dimension_semantics=("arbitrary",)),
    )(e0g, e0g, e0g, e0g, z1g, z1g, z1g, z1g, z2g, z2g, z2g, z2g,
      degg, degg, degg, degg)


def kernel(user_table, item_table, adj_val, adj_row, adj_col, user, positive, negative):
    del adj_val
    e0 = jnp.concatenate([user_table, item_table], axis=0)

    pad_r = jnp.full((PAD,), N_U, jnp.int32)
    pad_c = jnp.zeros((PAD,), jnp.int32)
    rows2d = jnp.concatenate(
        [adj_row[:EH], pad_r, adj_row[EH:] - N_U, pad_r]).reshape(ROWS2D, 128)
    cols2d = jnp.concatenate(
        [adj_col[:EH], pad_c, adj_col[EH:], pad_c]).reshape(ROWS2D, 128)

    degflat = _deg_kernel(rows2d).reshape(-1)
    deg2d = jnp.concatenate(
        [degflat[:N_U], degflat[HR * 128:HR * 128 + N_U]])[:, None]

    y0, degbc = _scale_a(e0, deg2d)
    y1 = _prop1_kernel(y0, rows2d, cols2d, degbc)

    idxn2d = jnp.concatenate(
        [user, positive + N_U, negative + N_U]).astype(jnp.int32).reshape(96, 128)
    idxacc2d = jnp.concatenate(
        [user, positive, negative]).astype(jnp.int32).reshape(96, 128)
    e0g, y1g, degg, z2g = _prop2_kernel(
        y1, rows2d, cols2d, e0, degbc, idxn2d, idxacc2d)

    out = _losses(e0g, y1g, z2g, degg)
    return out[0, :3]
